# baseline (speedup 1.0000x reference)
import functools

import jax
import jax.numpy as jnp
from jax import lax
from jax.experimental import pallas as pl
from jax.experimental.pallas import tpu as pltpu
from jax.experimental.pallas import tpu_sc as plsc

BUCKET_SIZE = 100000
EMBED_DIM = 64
EDGE_INPUT_DIM = 128
E_TOTAL = 320000
D_EDGE = 16

NC = 2
NS = 16
NW = NC * NS

GATHER_W = 80
BAND_W = 64
BAND_BASE = [((f + 1) * 131) // 8 * 8 for f in range(D_EDGE)]
CACHE_ROWS = D_EDGE * BAND_W


def _make_sc_pool(e_total, c):
    epw = e_total // NW
    iters = epw // c
    idx_per = c * D_EDGE
    ngath = idx_per // GATHER_W
    fb_q = 4
    fb_edges = c // fb_q
    fb_rows = fb_edges * D_EDGE
    fb_t = ngath // fb_q
    assert epw % c == 0 and idx_per % GATHER_W == 0 and ngath % fb_q == 0
    assert c % 8 == 0 and fb_rows >= BAND_W

    def sc_pool(feat_hbm, table_hbm, out_hbm, cache_v, feat_v, slots_v,
                idx_v, rows_v, pool_v, miss_s, fsem0, fsem1, psem0, psem1,
                gsem):
        wid = lax.axis_index("s") * NC + lax.axis_index("c")
        base0 = wid * epw
        offs = (lax.iota(jnp.int32, 16) + 1) * 131
        basevec = offs & jnp.int32(~7)
        bandoff = lax.iota(jnp.int32, 16) * BAND_W
        fsems = (fsem0, fsem1)
        psems = (psem0, psem1)

        for f in range(D_EDGE):
            pltpu.sync_copy(
                table_hbm.at[pl.ds(BAND_BASE[f], BAND_W), :],
                rows_v.at[pl.ds(0, BAND_W), :],
            )

            def pack_row(r, c2, f=f):
                for g in range(2):
                    a = rows_v[r, pl.ds(g * 32, 16)]
                    b = rows_v[r, pl.ds(g * 32 + 16, 16)]
                    cache_v[f * BAND_W + r, pl.ds(g * 32, 32)] = plsc.pack(
                        a, b, format=plsc.PackFormat.INTERLEAVED)
                return c2

            lax.fori_loop(0, BAND_W, pack_row, 0)

        def feat_slice(ci):
            return feat_hbm.at[pl.ds(base0 + ci * c, c), :]

        def out_slice(ci):
            return out_hbm.at[pl.ds(base0 + ci * c, c), :]

        def hash_chunk(ci, p):
            def hash_row(e, am):
                y = lax.abs(feat_v[p, e, :])
                n0 = y.astype(jnp.int32)
                fr = y - n0.astype(jnp.float32)
                inc = jnp.where(
                    fr > jnp.float32(0.5), jnp.int32(1),
                    jnp.where(fr == jnp.float32(0.5), n0 & 1, jnp.int32(0)))
                h = lax.rem(n0 + inc + offs, jnp.int32(BUCKET_SIZE))
                u = h - basevec
                miss = jnp.any((u < 0) | (u >= BAND_W))
                slots_v[p, e, :] = jnp.clip(u, 0, BAND_W - 1) + bandoff
                t = e // (GATHER_W // D_EDGE)
                col = (e % (GATHER_W // D_EDGE)) * D_EDGE
                idx_v[p, t, pl.ds(col, D_EDGE)] = h
                return jnp.logical_or(am, miss)

            any_miss = lax.fori_loop(0, c, hash_row, jnp.bool_(False))
            miss_s[p] = jnp.where(any_miss, jnp.int32(1), jnp.int32(0))

        def acc_chunk(ci, p):
            @plsc.parallel_loop(0, c, step=1, unroll=2)
            def _(e):
                sv = slots_v[p, e, :]
                sl = [sv[f] for f in range(D_EDGE)]
                for g in range(2):
                    va, vb = [], []
                    for f in range(D_EDGE):
                        w = cache_v[sl[f], pl.ds(g * 32, 32)]
                        a, b = plsc.unpack(
                            w, format=plsc.PackFormat.INTERLEAVED,
                            preferred_element_type=jnp.float32)
                        va.append(a)
                        vb.append(b)
                    for half, v in ((0, va), (1, vb)):
                        while len(v) > 1:
                            v = [v[2 * k] + v[2 * k + 1]
                                 for k in range(len(v) // 2)]
                        pool_v[p, e, pl.ds(g * 32 + half * 16, 16)] = v[0]

        def fallback_chunk(ci, p):
            for q in range(fb_q):
                for t in range(fb_t):
                    pltpu.async_copy(
                        table_hbm.at[idx_v.at[p, q * fb_t + t]],
                        rows_v.at[pl.ds(t * GATHER_W, GATHER_W), :],
                        gsem,
                    )
                for t in range(fb_t):
                    pltpu.make_async_copy(
                        table_hbm.at[idx_v.at[p, q * fb_t + t]],
                        rows_v.at[pl.ds(t * GATHER_W, GATHER_W), :],
                        gsem,
                    ).wait()

                def fb_edge(e, c2):
                    r0 = e * D_EDGE
                    for s in range(EMBED_DIM // 16):
                        cs = pl.ds(s * 16, 16)
                        v = [rows_v[r0 + f, cs] for f in range(D_EDGE)]
                        while len(v) > 1:
                            v = [v[2 * k] + v[2 * k + 1]
                                 for k in range(len(v) // 2)]
                        pool_v[p, q * fb_edges + e, cs] = v[0]
                    return c2

                lax.fori_loop(0, fb_edges, fb_edge, 0)

        pltpu.async_copy(feat_slice(jnp.int32(0)), feat_v.at[0], fsems[0])

        def body(i, carry):
            for b in range(2):
                ci = 2 * i + b
                nc_ = ci + 1

                pltpu.make_async_copy(feat_slice(ci), feat_v.at[b],
                                      fsems[b]).wait()

                @pl.when(nc_ < iters)
                def _():
                    pltpu.async_copy(feat_slice(nc_), feat_v.at[1 - b],
                                     fsems[1 - b])

                hash_chunk(ci, b)

                @pl.when(ci >= 2)
                def _():
                    pltpu.make_async_copy(pool_v.at[b], out_slice(ci),
                                          psems[b]).wait()

                acc_chunk(ci, b)

                @pl.when(miss_s[b] != 0)
                def _():
                    fallback_chunk(ci, b)

                pltpu.async_copy(pool_v.at[b], out_slice(ci), psems[b])
            return carry

        lax.fori_loop(0, iters // 2, body, 0)

        if iters % 2:
            ci = jnp.int32(iters - 1)
            pltpu.make_async_copy(feat_slice(ci), feat_v.at[0],
                                  fsems[0]).wait()
            hash_chunk(ci, 0)
            pltpu.make_async_copy(pool_v.at[0], out_slice(ci),
                                  psems[0]).wait()
            acc_chunk(ci, 0)

            @pl.when(miss_s[0] != 0)
            def _():
                fallback_chunk(ci, 0)

            pltpu.async_copy(pool_v.at[0], out_slice(ci), psems[0])

        b_last = (iters - 1) % 2
        pltpu.make_async_copy(pool_v.at[1 - b_last],
                              out_slice(jnp.int32(iters - 2)),
                              psems[1 - b_last]).wait()
        pltpu.make_async_copy(pool_v.at[b_last],
                              out_slice(jnp.int32(iters - 1)),
                              psems[b_last]).wait()

    return functools.partial(
        pl.kernel,
        mesh=plsc.VectorSubcoreMesh(core_axis_name="c", subcore_axis_name="s"),
        compiler_params=pltpu.CompilerParams(
            use_tc_tiling_on_sc=False, needs_layout_passes=False),
        out_type=jax.ShapeDtypeStruct((e_total, EMBED_DIM), jnp.float32),
        scratch_types=[
            pltpu.VMEM((CACHE_ROWS, EMBED_DIM), jnp.bfloat16),
            pltpu.VMEM((2, c, D_EDGE), jnp.float32),
            pltpu.VMEM((2, c, D_EDGE), jnp.int32),
            pltpu.VMEM((2, ngath, GATHER_W), jnp.int32),
            pltpu.VMEM((fb_rows, EMBED_DIM), jnp.float32),
            pltpu.VMEM((2, c, EMBED_DIM), jnp.float32),
            pltpu.SMEM((2,), jnp.int32),
            pltpu.SemaphoreType.DMA,
            pltpu.SemaphoreType.DMA,
            pltpu.SemaphoreType.DMA,
            pltpu.SemaphoreType.DMA,
            pltpu.SemaphoreType.DMA,
        ],
    )(sc_pool)


N_SPLIT = 2
E_HALF = E_TOTAL // N_SPLIT
_sc_pool_half = _make_sc_pool(E_HALF, 40)


def _mlp_body(x_ref, w1_ref, b1_ref, w2_ref, b2_ref, o_ref):
    x = (x_ref[...] * jnp.float32(1.0 / D_EDGE)).astype(jnp.bfloat16)
    h = jnp.dot(x, w1_ref[...].astype(jnp.bfloat16),
                preferred_element_type=jnp.float32)
    h = jnp.maximum(h + b1_ref[...], 0.0).astype(jnp.bfloat16)
    o = jnp.dot(h, w2_ref[...].astype(jnp.bfloat16),
                preferred_element_type=jnp.float32)
    o_ref[...] = o + b2_ref[...]


BE = 3200


def _mlp(pooled, W1, b1, W2, b2):
    n = pooled.shape[0]
    return pl.pallas_call(
        _mlp_body,
        grid=(n // BE,),
        in_specs=[
            pl.BlockSpec((BE, EMBED_DIM), lambda i: (i, 0)),
            pl.BlockSpec((EMBED_DIM, EDGE_INPUT_DIM), lambda i: (0, 0)),
            pl.BlockSpec((1, EDGE_INPUT_DIM), lambda i: (0, 0)),
            pl.BlockSpec((EDGE_INPUT_DIM, EDGE_INPUT_DIM), lambda i: (0, 0)),
            pl.BlockSpec((1, EDGE_INPUT_DIM), lambda i: (0, 0)),
        ],
        out_specs=pl.BlockSpec((BE, EDGE_INPUT_DIM), lambda i: (i, 0)),
        out_shape=jax.ShapeDtypeStruct((n, EDGE_INPUT_DIM), jnp.float32),
    )(pooled, W1, b1.reshape(1, -1), W2, b2.reshape(1, -1))


def kernel(edge_features, discrete_mask, emb_table, W1, b1, W2, b2):
    outs = []
    for s in range(N_SPLIT):
        feats = lax.slice_in_dim(edge_features, s * E_HALF, (s + 1) * E_HALF)
        pooled = _sc_pool_half(feats, emb_table)
        outs.append(_mlp(pooled, W1, b1, W2, b2))
    return jnp.concatenate(outs, axis=0)
